# baseline (speedup 1.0000x reference)
import functools

import jax
import jax.numpy as jnp
from jax import lax
from jax.experimental import pallas as pl
from jax.experimental.pallas import tpu as pltpu
from jax.experimental.pallas import tpu_sc as plsc

_B, _S, _D, _V = 128, 77, 256, 50000


_CH = 104
_NROW = 96


def _sc_gather(table, idx):
    info = plsc.get_sparse_core_info()
    nc, ns = info.num_cores, info.num_subcores
    nw = nc * ns
    n = _NROW * _CH
    per_w = n // nw
    nch = per_w // _CH

    mesh = plsc.VectorSubcoreMesh(core_axis_name="c", subcore_axis_name="s")

    @functools.partial(
        pl.kernel,
        mesh=mesh,
        out_type=jax.ShapeDtypeStruct((n, _D), jnp.float32),
        scratch_types=[
            pltpu.VMEM((per_w,), jnp.int32),
            pltpu.VMEM((per_w, _D), jnp.float32),
            pltpu.SemaphoreType.DMA,
        ],
    )
    def gather_k(table_hbm, idx_hbm, out_hbm, idx_v, rows_v, sem):
        wid = lax.axis_index("s") * nc + lax.axis_index("c")
        base = wid * per_w
        pltpu.sync_copy(idx_hbm.at[pl.ds(base, per_w)], idx_v)
        cps = [
            pltpu.async_copy(
                table_hbm.at[idx_v.at[pl.ds(c * _CH, _CH)]],
                rows_v.at[pl.ds(c * _CH, _CH)],
                sem,
            )
            for c in range(nch)
        ]
        for cp in cps:
            cp.wait()
        pltpu.sync_copy(rows_v, out_hbm.at[pl.ds(base, per_w)])

    return gather_k(table, idx)


def _rownorm(x):
    return jnp.maximum(jnp.sqrt(jnp.sum(x * x, axis=-1, keepdims=True)), 1e-15)


def _expmap0(u):
    n = _rownorm(u)
    return jnp.tanh(n) * u / n


def _logmap0(y):
    n = _rownorm(y)
    c = jnp.minimum(n, 1.0 - 1e-7)
    atanh = 0.5 * jnp.log((1.0 + c) / (1.0 - c))
    return atanh * y / n


def _dot_nt(a, w):
    return lax.dot_general(a, w, (((1,), (1,)), ((), ())),
                           preferred_element_type=jnp.float32)


def _tc_body(*refs):
    emb_ref, edge_ref, pos_ref = refs[0:3]
    wrefs = refs[3:8]
    brefs = refs[8:13]
    wrel_ref, wroot_ref, bg_ref, out_ref = refs[13:17]

    x = emb_ref[0] + pos_ref[...]
    for i in range(5):
        x = _dot_nt(x, wrefs[i][...]) + brefs[i][...]
        if i < 4:
            x = x * jax.nn.sigmoid(x)
    xt = _logmap0(_expmap0(x))
    adj = (edge_ref[0] != 0).astype(jnp.float32)
    msg = lax.dot_general(adj, xt, (((0,), (0,)), ((), ())),
                          preferred_element_type=jnp.float32)
    out_t = _dot_nt(msg, wrel_ref[...]) + _dot_nt(xt, wroot_ref[...]) + bg_ref[...]
    y = _logmap0(_expmap0(out_t))
    y = jnp.where(y >= 0.0, y, 0.01 * y)
    res = _logmap0(_expmap0(y))
    out_ref[...] = res[None]


def _tc_forward(emb3, edge, pos, proj_w, proj_b, wrel, wroot, bg):
    full2 = lambda shape: pl.BlockSpec(shape, lambda i: (0,) * len(shape))
    in_specs = (
        [
            pl.BlockSpec((1, _S, _D), lambda i: (i, 0, 0)),
            pl.BlockSpec((1, _S, _S), lambda i: (i, 0, 0)),
            full2((_S, _D)),
        ]
        + [full2((_D, _D)) for _ in range(5)]
        + [full2((1, _D)) for _ in range(5)]
        + [full2((_D, _D)), full2((_D, _D)), full2((1, _D))]
    )
    return pl.pallas_call(
        _tc_body,
        grid=(_B,),
        in_specs=in_specs,
        out_specs=pl.BlockSpec((1, _S, _D), lambda i: (i, 0, 0)),
        out_shape=jax.ShapeDtypeStruct((_B, _S, _D), jnp.float32),
    )(emb3, edge, pos, *proj_w, *[b[None] for b in proj_b], wrel, wroot, bg[None])


def kernel(params, tokens, edge):
    idx = tokens.astype(jnp.int32).reshape(-1)
    idx = jnp.concatenate([idx, jnp.zeros((_NROW * _CH - _B * _S,), jnp.int32)])
    emb = _sc_gather(params["token_table"], idx)
    emb3 = emb[: _B * _S].reshape(_B, _S, _D)
    return _tc_forward(
        emb3,
        edge,
        params["pos_table"],
        params["proj_W"],
        params["proj_b"],
        params["gcn_Wrel"][3],
        params["gcn_Wroot"][3],
        params["gcn_b"][3],
    )


# speedup vs baseline: 2.6680x; 2.6680x over previous
import functools

import jax
import jax.numpy as jnp
from jax import lax
from jax.experimental import pallas as pl
from jax.experimental.pallas import tpu as pltpu
from jax.experimental.pallas import tpu_sc as plsc

_B, _S, _D, _V = 128, 77, 256, 50000
_SP = 80
_BB = 8


def _sc_gather(table, idx):
    info = plsc.get_sparse_core_info()
    nc, ns = info.num_cores, info.num_subcores
    nw = nc * ns
    n = _B * _SP
    per_w = n // nw
    nch = per_w // _SP

    mesh = plsc.VectorSubcoreMesh(core_axis_name="c", subcore_axis_name="s")

    @functools.partial(
        pl.kernel,
        mesh=mesh,
        out_type=jax.ShapeDtypeStruct((n, _D), jnp.float32),
        scratch_types=[
            pltpu.VMEM((per_w,), jnp.int32),
            pltpu.VMEM((per_w, _D), jnp.float32),
            pltpu.SemaphoreType.DMA,
        ],
    )
    def gather_k(table_hbm, idx_hbm, out_hbm, idx_v, rows_v, sem):
        wid = lax.axis_index("s") * nc + lax.axis_index("c")
        base = wid * per_w
        pltpu.sync_copy(idx_hbm.at[pl.ds(base, per_w)], idx_v)
        cps = [
            pltpu.async_copy(
                table_hbm.at[idx_v.at[pl.ds(c * _SP, _SP)]],
                rows_v.at[pl.ds(c * _SP, _SP)],
                sem,
            )
            for c in range(nch)
        ]
        for cp in cps:
            cp.wait()
        pltpu.sync_copy(rows_v, out_hbm.at[pl.ds(base, per_w)])

    return gather_k(table, idx)


def _rownorm(x):
    return jnp.maximum(jnp.sqrt(jnp.sum(x * x, axis=-1, keepdims=True)), 1e-15)


def _expmap0(u):
    n = _rownorm(u)
    return jnp.tanh(n) * u / n


def _logmap0(y):
    n = _rownorm(y)
    c = jnp.minimum(n, 1.0 - 1e-7)
    atanh = 0.5 * jnp.log((1.0 + c) / (1.0 - c))
    return atanh * y / n


def _dot_nt(a, w):
    return lax.dot_general(a, w, (((1,), (1,)), ((), ())),
                           preferred_element_type=jnp.float32)


def _tc_body(*refs):
    emb_ref, edge_ref, pos_ref = refs[0:3]
    wrefs = refs[3:8]
    brefs = refs[8:13]
    wrel_ref, wroot_ref, bg_ref, out_ref = refs[13:17]

    x3 = emb_ref[...] + pos_ref[...][None]
    x = x3.reshape(_BB * _SP, _D)
    for i in range(5):
        x = _dot_nt(x, wrefs[i][...]) + brefs[i][...]
        if i < 4:
            x = x * jax.nn.sigmoid(x)
    xt = _logmap0(_expmap0(x))
    xt3 = xt.reshape(_BB, _SP, _D)
    adj = (edge_ref[...] != 0).astype(jnp.float32)
    msg3 = jnp.stack(
        [
            lax.dot_general(adj[b], xt3[b], (((0,), (0,)), ((), ())),
                            preferred_element_type=jnp.float32)
            for b in range(_BB)
        ]
    )
    msg = msg3.reshape(_BB * _SP, _D)
    out_t = _dot_nt(msg, wrel_ref[...]) + _dot_nt(xt, wroot_ref[...]) + bg_ref[...]
    y = _logmap0(_expmap0(out_t))
    y = jnp.where(y >= 0.0, y, 0.01 * y)
    res = _logmap0(_expmap0(y))
    out_ref[...] = res.reshape(_BB, _SP, _D)


def _tc_forward(emb3, edge3, pos, proj_w, proj_b, wrel, wroot, bg):
    full2 = lambda shape: pl.BlockSpec(shape, lambda i: (0,) * len(shape))
    in_specs = (
        [
            pl.BlockSpec((_BB, _SP, _D), lambda i: (i, 0, 0)),
            pl.BlockSpec((_BB, _SP, _SP), lambda i: (i, 0, 0)),
            full2((_SP, _D)),
        ]
        + [full2((_D, _D)) for _ in range(5)]
        + [full2((1, _D)) for _ in range(5)]
        + [full2((_D, _D)), full2((_D, _D)), full2((1, _D))]
    )
    return pl.pallas_call(
        _tc_body,
        grid=(_B // _BB,),
        in_specs=in_specs,
        out_specs=pl.BlockSpec((_BB, _SP, _D), lambda i: (i, 0, 0)),
        out_shape=jax.ShapeDtypeStruct((_B, _SP, _D), jnp.float32),
    )(emb3, edge3, pos, *proj_w, *[b[None] for b in proj_b], wrel, wroot, bg[None])


def kernel(params, tokens, edge):
    tok80 = jnp.pad(tokens.astype(jnp.int32), ((0, 0), (0, _SP - _S)))
    emb = _sc_gather(params["token_table"], tok80.reshape(-1))
    emb3 = emb.reshape(_B, _SP, _D)
    edge80 = jnp.pad(edge, ((0, 0), (0, _SP - _S), (0, _SP - _S)))
    pos80 = jnp.pad(params["pos_table"], ((0, _SP - _S), (0, 0)))
    out80 = _tc_forward(
        emb3,
        edge80,
        pos80,
        params["proj_W"],
        params["proj_b"],
        params["gcn_Wrel"][3],
        params["gcn_Wroot"][3],
        params["gcn_b"][3],
    )
    return out80[:, :_S, :]
